# baseline (speedup 1.0000x reference)
import jax
import jax.numpy as jnp
from jax.experimental import pallas as pl
from jax.experimental.pallas import tpu as pltpu


_IMGS_PER_STEP = 16


def _matmul_bias_relu_kernel(w_ref, p_ref, b_ref, o_ref):
    w = w_ref[...]
    b = b_ref[...]
    for i in range(_IMGS_PER_STEP):
        acc = jnp.dot(w, p_ref[i], preferred_element_type=jnp.float32)
        o_ref[i] = jnp.maximum(acc + b, 0.0).astype(o_ref.dtype)


def kernel(x, weight, bias):
    N, Cin, H, W = x.shape
    Cout = weight.shape[0]
    k = 16
    Ho, Wo = H // k, W // k
    M = Ho * Wo
    K = Cin * k * k

    M_pad = 256
    patches = (
        x.reshape(N, Cin, Ho, k, Wo, k)
        .transpose(0, 1, 3, 5, 2, 4)
        .reshape(N, K, M)
        .astype(jnp.bfloat16)
    )
    patches = jnp.pad(patches, ((0, 0), (0, 0), (0, M_pad - M)))
    w_mat = weight.reshape(Cout, K).astype(jnp.bfloat16)
    b_col = bias.reshape(Cout, 1)

    out = pl.pallas_call(
        _matmul_bias_relu_kernel,
        out_shape=jax.ShapeDtypeStruct((N, Cout, M_pad), x.dtype),
        grid_spec=pl.GridSpec(
            grid=(N // _IMGS_PER_STEP,),
            in_specs=[
                pl.BlockSpec((Cout, K), lambda n: (0, 0)),
                pl.BlockSpec((_IMGS_PER_STEP, K, M_pad), lambda n: (n, 0, 0)),
                pl.BlockSpec((Cout, 1), lambda n: (0, 0)),
            ],
            out_specs=pl.BlockSpec((_IMGS_PER_STEP, Cout, M_pad),
                                   lambda n: (n, 0, 0)),
        ),
        compiler_params=pltpu.CompilerParams(
            dimension_semantics=("arbitrary",)),
    )(w_mat, patches, b_col)

    return out[:, :, :M].reshape(N, Cout, Ho, Wo)
